# Initial kernel scaffold; baseline (speedup 1.0000x reference)
import jax
import jax.numpy as jnp
from jax.experimental import pallas as pl


def kernel(locs_3d, x_os_3d, y_os_3d, z_os_3d, ints_3d, sigma, scale):
    raise NotImplementedError("write your pallas kernel here")


# speedup vs baseline: 1088.2081x; 1088.2081x over previous
import functools

import jax
import jax.numpy as jnp
from jax import lax
from jax.experimental import pallas as pl
from jax.experimental.pallas import tpu as pltpu
from jax.experimental.pallas import tpu_sc as plsc

H = 96
W = 96
D = 32
HPAD = 128
WP = 108
DROW = 64
NVOX = H * W * D
NSUB = 16
ROWSTAGE = W * D
NGROUP = ROWSTAGE // 256
SLABW = 8 * WP * DROW
OUTN = HPAD * WP * DROW


def _sum16(v):
  s = v[0]
  for k in range(1, 16):
    s = s + v[k]
  return s


def _sum9(v):
  s = v[0]
  for k in range(1, 9):
    s = s + v[k]
  return s


def _sc_body(locs_h, xos_h, yos_h, zos_h, ints_h, par_h, out_h,
             vol, lbuf, xbuf, ybuf, zbuf, ibuf, pbuf):
  c = lax.axis_index("c")
  s = lax.axis_index("s")
  ii = lax.iota(jnp.int32, 16)
  fi = ii.astype(jnp.float32)
  z16f = jnp.zeros((16,), jnp.float32)
  mask9 = jnp.where(ii < 9, 1.0, 0.0)
  gf = (fi - 4.0) * mask9

  pltpu.sync_copy(par_h, pbuf)
  pvec = pbuf[...]
  pinv_s = pvec[0]
  amp_s = pvec[1]

  def _zv(j, _):
    vol[pl.ds(j * 16, 16)] = z16f
    return 0
  lax.fori_loop(0, SLABW // 16, _zv, 0)

  hp0 = 8 * s

  def _hrow(j, _):
    h = hp0 - 10 + j

    @pl.when(jnp.logical_and(h >= 0, h < H))
    def _():
      off = h * ROWSTAGE
      pltpu.sync_copy(locs_h.at[c, pl.ds(off, ROWSTAGE)], lbuf)
      pltpu.sync_copy(xos_h.at[c, pl.ds(off, ROWSTAGE)], xbuf)
      pltpu.sync_copy(yos_h.at[c, pl.ds(off, ROWSTAGE)], ybuf)
      pltpu.sync_copy(zos_h.at[c, pl.ds(off, ROWSTAGE)], zbuf)
      pltpu.sync_copy(ints_h.at[c, pl.ds(off, ROWSTAGE)], ibuf)

      def _group(g, _):
        gbase = g * 256
        acc = lbuf[pl.ds(gbase, 16)]
        for t in range(1, 16):
          acc = acc + lbuf[pl.ds(gbase + t * 16, 16)]
        gsum = _sum16(acc)

        @pl.when(gsum > 0.0)
        def _():
          def _chunk(i2, _):
            cbase = gbase + i2 * 16
            lv = lbuf[pl.ds(cbase, 16)]
            csum = _sum16(lv)

            @pl.when(csum > 0.0)
            def _():
              xv = xbuf[pl.ds(cbase, 16)]
              yv = ybuf[pl.ds(cbase, 16)]
              zv = zbuf[pl.ds(cbase, 16)]
              iv = ibuf[pl.ds(cbase, 16)]

              def _lane(e, _):
                sel = jnp.where(ii == e, 1.0, 0.0)
                le = _sum16(lv * sel)

                @pl.when(le > 0.0)
                def _():
                  xe = _sum16(xv * sel)
                  ye = _sum16(yv * sel)
                  ze = _sum16(zv * sel)
                  inte = _sum16(iv * sel)
                  r = cbase + e
                  d_e = lax.rem(r, D)
                  wcol = lax.div(r, D)

                  ax = jnp.exp(-(gf - xe) * (gf - xe) * pinv_s) * mask9
                  ay = jnp.exp(-(gf - ye) * (gf - ye) * pinv_s) * mask9
                  az = jnp.exp(-(gf - ze) * (gf - ze) * pinv_s) * mask9
                  denv = z16f + (_sum9(ax) * _sum9(ay) * _sum9(az))
                  numv = z16f + inte * amp_s
                  azw = az * (numv / denv)

                  s_off = d_e + 2
                  segs = []
                  for k in range(3):
                    idxk = ii + (16 * k - s_off)
                    vmask = jnp.where(idxk >= 0, 1.0, 0.0) * jnp.where(
                        idxk <= 8, 1.0, 0.0)
                    idxc = jnp.minimum(jnp.maximum(idxk, 0), 15)
                    segs.append(
                        azw.at[idxc].get(mode="promise_in_bounds") * vmask)

                  for ox in range(9):
                    hp = h + 2 + ox
                    hpl = hp - hp0

                    @pl.when(jnp.logical_and(hpl >= 0, hpl < 8))
                    def _(ox=ox, hpl=hpl):
                      axc = ax[ox]
                      rowbase = (hpl * WP + wcol + 2) * DROW
                      for oy in range(9):
                        coef = axc * ay[oy]
                        ob = rowbase + oy * DROW
                        for k in range(3):
                          o = ob + 16 * k
                          vol[pl.ds(o, 16)] = (
                              vol[pl.ds(o, 16)] + segs[k] * coef)
                return 0
              lax.fori_loop(0, 16, _lane, 0)
            return 0
          lax.fori_loop(0, 16, _chunk, 0)
        return 0
      lax.fori_loop(0, NGROUP, _group, 0)
    return 0
  lax.fori_loop(0, 16, _hrow, 0)

  pltpu.sync_copy(vol, out_h.at[c, pl.ds(s * SLABW, SLABW)])


def _sc_call(locs2, x2, y2, z2, i2, par):
  mesh = plsc.VectorSubcoreMesh(core_axis_name="c", subcore_axis_name="s")
  f = functools.partial(
      pl.kernel,
      out_type=jax.ShapeDtypeStruct((2, OUTN), jnp.float32),
      mesh=mesh,
      scratch_types=[
          pltpu.VMEM((SLABW,), jnp.float32),
          pltpu.VMEM((ROWSTAGE,), jnp.float32),
          pltpu.VMEM((ROWSTAGE,), jnp.float32),
          pltpu.VMEM((ROWSTAGE,), jnp.float32),
          pltpu.VMEM((ROWSTAGE,), jnp.float32),
          pltpu.VMEM((ROWSTAGE,), jnp.float32),
          pltpu.VMEM((16,), jnp.float32),
      ],
  )(_sc_body)
  return f(locs2, x2, y2, z2, i2, par)


def kernel(locs_3d, x_os_3d, y_os_3d, z_os_3d, ints_3d, sigma, scale):
  locs2 = locs_3d.reshape(2, NVOX)
  x2 = x_os_3d.reshape(2, NVOX)
  y2 = y_os_3d.reshape(2, NVOX)
  z2 = z_os_3d.reshape(2, NVOX)
  i2 = ints_3d.reshape(2, NVOX)
  inv2s2 = 1.0 / (2.0 * sigma.astype(jnp.float32) ** 2)
  amp = 1000.0 * scale.astype(jnp.float32)
  par = jnp.zeros((16,), jnp.float32).at[0].set(inv2s2).at[1].set(amp)
  vol = _sc_call(locs2, x2, y2, z2, i2, par)
  vol = vol.reshape(2, HPAD, WP, DROW)[:, 6:102, 6:102, 6:38]
  return vol.reshape(2, 1, H, W, D)
